# Initial kernel scaffold; baseline (speedup 1.0000x reference)
import functools

import jax
import jax.numpy as jnp
from jax import lax
from jax.experimental import pallas as pl
from jax.experimental.pallas import tpu as pltpu
from jax.experimental.pallas import tpu_sc as plsc

NUM_FEATURES = 26
VOCAB = 16
LANES = 16
PERIOD = NUM_FEATURES * LANES // 2
VREGS_PER_PERIOD = PERIOD // LANES


@functools.lru_cache(maxsize=None)
def _make_lookup(total: int):
    info = plsc.get_sparse_core_info()
    nw = info.num_cores * info.num_subcores
    assert total % (nw * PERIOD) == 0
    chunk = total // nw
    groups = chunk // PERIOD

    mesh = plsc.VectorSubcoreMesh(core_axis_name="c", subcore_axis_name="s")

    @functools.partial(
        pl.kernel,
        mesh=mesh,
        out_type=jax.ShapeDtypeStruct((total,), jnp.float32),
        scratch_types=[
            pltpu.VMEM((chunk,), jnp.int32),
            pltpu.VMEM((chunk,), jnp.float32),
            pltpu.VMEM((NUM_FEATURES * VOCAB,), jnp.float32),
            pltpu.VMEM((PERIOD,), jnp.int32),
        ],
    )
    def lookup(tok_hbm, tbl_hbm, out_hbm, tok_v, out_v, tbl_v, offs_v):
        wid = lax.axis_index("s") * info.num_cores + lax.axis_index("c")
        base = wid * chunk
        pltpu.sync_copy(tok_hbm.at[pl.ds(base, chunk)], tok_v)
        pltpu.sync_copy(tbl_hbm, tbl_v)
        for j in range(VREGS_PER_PERIOD):
            p = lax.iota(jnp.int32, (LANES,)) + (j * LANES)
            offs_v[pl.ds(j * LANES, LANES)] = lax.rem(p, NUM_FEATURES) * VOCAB

        def body(g, carry):
            go = g * PERIOD
            for j in range(VREGS_PER_PERIOD):
                o = go + j * LANES
                idx = tok_v[pl.ds(o, LANES)] + offs_v[pl.ds(j * LANES, LANES)]
                out_v[pl.ds(o, LANES)] = plsc.load_gather(tbl_v, [idx])
            return carry

        lax.fori_loop(0, groups, body, 0)
        pltpu.sync_copy(out_v, out_hbm.at[pl.ds(base, chunk)])

    return lookup


def kernel(inputs, mapping):
    shape = inputs.shape
    tok = inputs.astype(jnp.int32).reshape(-1)
    tbl = mapping.astype(jnp.float32).reshape(-1)
    out = _make_lookup(tok.size)(tok, tbl)
    return out.reshape(shape)


# speedup vs baseline: 58.3764x; 58.3764x over previous
import functools

import jax
import jax.numpy as jnp
from jax import lax
from jax.experimental import pallas as pl
from jax.experimental.pallas import tpu as pltpu
from jax.experimental.pallas import tpu_sc as plsc

NUM_FEATURES = 26
VOCAB = 16
LANES = 16
PERIOD = NUM_FEATURES * LANES // 2
VREGS_PER_PERIOD = PERIOD // LANES


@functools.lru_cache(maxsize=None)
def _make_lookup(total: int):
    info = plsc.get_sparse_core_info()
    nw = info.num_cores * info.num_subcores
    assert total % (nw * PERIOD) == 0
    chunk = total // nw
    groups = chunk // PERIOD

    mesh = plsc.VectorSubcoreMesh(core_axis_name="c", subcore_axis_name="s")

    @functools.partial(
        pl.kernel,
        mesh=mesh,
        out_type=jax.ShapeDtypeStruct((total,), jnp.float32),
        scratch_types=[
            pltpu.VMEM((chunk,), jnp.int32),
            pltpu.VMEM((chunk,), jnp.float32),
            pltpu.VMEM((NUM_FEATURES * VOCAB,), jnp.float32),
            pltpu.VMEM((PERIOD,), jnp.int32),
        ],
        compiler_params=pltpu.CompilerParams(needs_layout_passes=False),
    )
    def lookup(tok_hbm, tbl_hbm, out_hbm, tok_v, out_v, tbl_v, offs_v):
        wid = lax.axis_index("s") * info.num_cores + lax.axis_index("c")
        base = wid * chunk
        pltpu.sync_copy(tok_hbm.at[pl.ds(base, chunk)], tok_v)
        pltpu.sync_copy(tbl_hbm, tbl_v)
        for j in range(VREGS_PER_PERIOD):
            p = lax.iota(jnp.int32, LANES) + (j * LANES)
            offs_v[pl.ds(j * LANES, LANES)] = lax.rem(p, NUM_FEATURES) * VOCAB

        def body(g, carry):
            go = g * PERIOD
            for j in range(VREGS_PER_PERIOD):
                o = go + j * LANES
                idx = tok_v[pl.ds(o, LANES)] + offs_v[pl.ds(j * LANES, LANES)]
                out_v[pl.ds(o, LANES)] = plsc.load_gather(tbl_v, [idx])
            return carry

        lax.fori_loop(0, groups, body, 0)
        pltpu.sync_copy(out_v, out_hbm.at[pl.ds(base, chunk)])

    return lookup


def kernel(inputs, mapping):
    shape = inputs.shape
    tok = inputs.astype(jnp.int32).reshape(-1)
    tbl = mapping.astype(jnp.float32).reshape(-1)
    out = _make_lookup(tok.size)(tok, tbl)
    return out.reshape(shape)
